# baseline (speedup 1.0000x reference)
import functools

import jax
import jax.numpy as jnp
from jax import lax
from jax.experimental import pallas as pl
from jax.experimental.pallas import tpu as pltpu
from jax.experimental.pallas import tpu_sc as plsc

B, C, H, W = 4, 64, 384, 384
L = 16
NC, NS = 2, 16
NW = NC * NS
ROWS = B * H
RPW = ROWS // NW
R = 4
CHUNKS = RPW // R
WPB = H // RPW

_mesh = plsc.VectorSubcoreMesh(core_axis_name="c", subcore_axis_name="s")


@functools.partial(
    pl.kernel,
    mesh=_mesh,
    out_type=jax.ShapeDtypeStruct((B * C * H * W,), jnp.float32),
    scratch_types=[
        pltpu.VMEM((R * W,), jnp.int32),
        pltpu.VMEM((C * R * W,), jnp.float32),
        pltpu.SemaphoreType.DMA,
    ],
    compiler_params=pltpu.CompilerParams(
        use_tc_tiling_on_sc=False, needs_layout_passes=False),
)
def _one_hot_sc(x_hbm, out_hbm, x_v, buf, sem):
    wid = lax.axis_index("s") * NC + lax.axis_index("c")
    b = wid // WPB
    i_base = (wid % WPB) * RPW

    zeros = jnp.zeros((L,), jnp.float32)
    ones = jnp.ones((L,), jnp.float32)

    def _zero(t, carry):
        buf[pl.ds(t * L, L)] = zeros
        return carry
    lax.fori_loop(0, C * R * W // L, _zero, 0)

    def _scatter(val):
        def _p(p, carry):
            xv = x_v[pl.ds(p * L, L)]
            flat = xv * (R * W) + p * L + lax.iota(jnp.int32, L)
            plsc.store_scatter(buf, [flat], val)
            return carry
        lax.fori_loop(0, R * W // L, _p, 0)

    def _chunk(k, carry):
        row0 = (wid * RPW + k * R) * W
        i0 = i_base + k * R
        pltpu.sync_copy(x_hbm.at[pl.ds(row0, R * W)], x_v)
        _scatter(ones)
        copies = []
        for c in range(C):
            dst = ((b * C + c) * H + i0) * W
            copies.append(pltpu.async_copy(
                buf.at[pl.ds(c * R * W, R * W)],
                out_hbm.at[pl.ds(dst, R * W)], sem))
        for cp in copies:
            cp.wait()
        _scatter(zeros)
        return carry
    lax.fori_loop(0, CHUNKS, _chunk, 0)


def kernel(x):
    return _one_hot_sc(x.reshape(B * H * W)).reshape(B, C, H, W)
